# baseline (speedup 1.0000x reference)
import functools

import jax
import jax.numpy as jnp
from jax import lax
from jax.experimental import pallas as pl
from jax.experimental.pallas import tpu as pltpu
from jax.experimental.pallas import tpu_sc as plsc

EMB_DIM = 128
_NC, _NS = 2, 16
_NW = _NC * _NS
_CH = 128

_BLK = 2048
_NB = (5, 5, 10, 30)


def _table_body(e0, e1, e2, e3, w1, w2, w3, o):
    i = pl.program_id(0)

    @pl.when(i < 5)
    def _():
        o[...] = e0[...]

    @pl.when((i >= 5) & (i < 10))
    def _():
        o[...] = jnp.dot(e1[...], w1[...], preferred_element_type=jnp.float32)

    @pl.when((i >= 10) & (i < 20))
    def _():
        o[...] = jnp.dot(e2[...], w2[...], preferred_element_type=jnp.float32)

    @pl.when(i >= 20)
    def _():
        o[...] = jnp.dot(e3[...], w3[...], preferred_element_type=jnp.float32)


def _build_table(emb0, emb1, emb2, emb3, w1, w2, w3):
    nrows = _BLK * sum(_NB)
    return pl.pallas_call(
        _table_body,
        grid=(sum(_NB),),
        in_specs=[
            pl.BlockSpec((_BLK, 128), lambda i: (jnp.clip(i, 0, 4), 0)),
            pl.BlockSpec((_BLK, 64), lambda i: (jnp.clip(i - 5, 0, 4), 0)),
            pl.BlockSpec((_BLK, 32), lambda i: (jnp.clip(i - 10, 0, 9), 0)),
            pl.BlockSpec((_BLK, 16), lambda i: (jnp.clip(i - 20, 0, 29), 0)),
            pl.BlockSpec((64, EMB_DIM), lambda i: (0, 0)),
            pl.BlockSpec((32, EMB_DIM), lambda i: (0, 0)),
            pl.BlockSpec((16, EMB_DIM), lambda i: (0, 0)),
        ],
        out_specs=pl.BlockSpec((_BLK, EMB_DIM), lambda i: (i, 0)),
        out_shape=jax.ShapeDtypeStruct((nrows, EMB_DIM), jnp.float32),
    )(emb0, emb1, emb2, emb3, w1, w2, w3)


def _sc_lookup(x3, table):
    nch = x3.shape[1]
    n_out = _NW * nch * _CH
    mesh = plsc.VectorSubcoreMesh(core_axis_name="c", subcore_axis_name="s")

    nbuf = 5
    @functools.partial(
        pl.kernel,
        out_type=jax.ShapeDtypeStruct((n_out, EMB_DIM), jnp.float32),
        mesh=mesh,
        scratch_types=[
            pltpu.VMEM((nch, _CH), jnp.int32),
            pltpu.VMEM((nbuf, _CH, EMB_DIM), jnp.float32),
            pltpu.SemaphoreType.DMA,
            pltpu.SemaphoreType.DMA,
            pltpu.SemaphoreType.DMA,
            pltpu.SemaphoreType.DMA,
            pltpu.SemaphoreType.DMA,
        ],
    )
    def k(x_hbm, table_hbm, out_hbm, idx_v, rows_v, *sems):
        wid = lax.axis_index("s") * _NC + lax.axis_index("c")
        cbase = wid * nch
        pltpu.sync_copy(x_hbm.at[wid], idx_v)

        def to_rows(r, carry):
            for c in range(_CH // 16):
                v = idx_v[r, pl.ds(c * 16, 16)]
                shift = (jnp.where(v > 10000, 240, 0)
                         + jnp.where(v > 20000, 240, 0)
                         + jnp.where(v > 40000, 480, 0))
                idx_v[r, pl.ds(c * 16, 16)] = jnp.where(v == 0, 10000,
                                                        v + shift)
            return carry

        lax.fori_loop(0, nch, to_rows, 0)

        def fire(j, b):
            pltpu.async_copy(table_hbm.at[idx_v.at[j]], rows_v.at[b], sems[b])

        def wait(j, b):
            pltpu.make_async_copy(table_hbm.at[idx_v.at[j]], rows_v.at[b],
                                  sems[b]).wait()

        def store(j, b):
            pltpu.sync_copy(rows_v.at[b],
                            out_hbm.at[pl.ds((cbase + j) * _CH, _CH)])

        for b in range(nbuf):
            fire(b, b)

        def body(s, carry):
            for b in range(nbuf):
                j = nbuf * s + b
                wait(j, b)
                store(j, b)
                fire(jnp.minimum(j + nbuf, nch - 1), b)
            return carry

        lax.fori_loop(0, nch // nbuf, body, 0)
        for b in range(nbuf):
            wait(nch - 1, b)

    return k(x3, table)


def _re_body(i_ref, o_ref):
    for r in range(8):
        o_ref[r] = i_ref[0, pl.ds(r * 50, 50), :]


def _relayout(flat, nrow, ntok):
    grp = 8
    f3 = flat.reshape(nrow // grp, grp * ntok, EMB_DIM)
    return pl.pallas_call(
        _re_body,
        grid=(nrow // grp,),
        in_specs=[pl.BlockSpec((1, grp * ntok, EMB_DIM), lambda i: (i, 0, 0))],
        out_specs=pl.BlockSpec((grp, ntok, EMB_DIM), lambda i: (i, 0, 0)),
        out_shape=jax.ShapeDtypeStruct((nrow, ntok, EMB_DIM), jnp.float32),
    )(f3)


def kernel(x, emb0, emb1, emb2, emb3, W1, W2, W3):
    table = _build_table(emb0, emb1, emb2, emb3, W1, W2, W3)
    x3 = x.reshape(_NW, -1, _CH).astype(jnp.int32)
    out = _sc_lookup(x3, table)
    return _relayout(out, x.shape[0], x.shape[1])


# speedup vs baseline: 1.5007x; 1.5007x over previous
import functools

import jax
import jax.numpy as jnp
from jax import lax
from jax.experimental import pallas as pl
from jax.experimental.pallas import tpu as pltpu
from jax.experimental.pallas import tpu_sc as plsc

EMB_DIM = 128
_NC, _NS = 2, 16
_NW = _NC * _NS
_CH = 128

_BLK = 2048
_NB = (5, 5, 10, 30)


def _table_body(e0, e1, e2, e3, w1, w2, w3, o):
    i = pl.program_id(0)

    @pl.when(i < 5)
    def _():
        o[...] = e0[...]

    @pl.when((i >= 5) & (i < 10))
    def _():
        o[...] = jnp.dot(e1[...], w1[...], preferred_element_type=jnp.float32)

    @pl.when((i >= 10) & (i < 20))
    def _():
        o[...] = jnp.dot(e2[...], w2[...], preferred_element_type=jnp.float32)

    @pl.when(i >= 20)
    def _():
        o[...] = jnp.dot(e3[...], w3[...], preferred_element_type=jnp.float32)


def _build_table(emb0, emb1, emb2, emb3, w1, w2, w3):
    nrows = _BLK * sum(_NB)
    return pl.pallas_call(
        _table_body,
        grid=(sum(_NB),),
        in_specs=[
            pl.BlockSpec((_BLK, 128), lambda i: (jnp.clip(i, 0, 4), 0)),
            pl.BlockSpec((_BLK, 64), lambda i: (jnp.clip(i - 5, 0, 4), 0)),
            pl.BlockSpec((_BLK, 32), lambda i: (jnp.clip(i - 10, 0, 9), 0)),
            pl.BlockSpec((_BLK, 16), lambda i: (jnp.clip(i - 20, 0, 29), 0)),
            pl.BlockSpec((64, EMB_DIM), lambda i: (0, 0)),
            pl.BlockSpec((32, EMB_DIM), lambda i: (0, 0)),
            pl.BlockSpec((16, EMB_DIM), lambda i: (0, 0)),
        ],
        out_specs=pl.BlockSpec((_BLK, EMB_DIM), lambda i: (i, 0)),
        out_shape=jax.ShapeDtypeStruct((nrows, EMB_DIM), jnp.float32),
    )(emb0, emb1, emb2, emb3, w1, w2, w3)


def _sc_lookup(x3, table):
    nch = x3.shape[1]
    n_out = _NW * nch * _CH
    mesh = plsc.VectorSubcoreMesh(core_axis_name="c", subcore_axis_name="s")

    nbuf = 5
    @functools.partial(
        pl.kernel,
        out_type=jax.ShapeDtypeStruct((n_out, EMB_DIM), jnp.float32),
        mesh=mesh,
        scratch_types=[
            pltpu.VMEM((nch, _CH), jnp.int32),
            pltpu.VMEM((nbuf, _CH, EMB_DIM), jnp.float32),
            pltpu.SemaphoreType.DMA,
            pltpu.SemaphoreType.DMA,
            pltpu.SemaphoreType.DMA,
            pltpu.SemaphoreType.DMA,
            pltpu.SemaphoreType.DMA,
        ],
    )
    def k(x_hbm, table_hbm, out_hbm, idx_v, rows_v, *sems):
        wid = lax.axis_index("s") * _NC + lax.axis_index("c")
        cbase = wid * nch
        pltpu.sync_copy(x_hbm.at[wid], idx_v)

        def to_rows(r, carry):
            for c in range(_CH // 16):
                v = idx_v[r, pl.ds(c * 16, 16)]
                shift = (jnp.where(v > 10000, 240, 0)
                         + jnp.where(v > 20000, 240, 0)
                         + jnp.where(v > 40000, 480, 0))
                idx_v[r, pl.ds(c * 16, 16)] = jnp.where(v == 0, 10000,
                                                        v + shift)
            return carry

        lax.fori_loop(0, nch, to_rows, 0)

        def fire(j, b):
            pltpu.async_copy(table_hbm.at[idx_v.at[j]], rows_v.at[b], sems[b])

        def wait(j, b):
            pltpu.make_async_copy(table_hbm.at[idx_v.at[j]], rows_v.at[b],
                                  sems[b]).wait()

        def store(j, b):
            pltpu.sync_copy(rows_v.at[b],
                            out_hbm.at[pl.ds((cbase + j) * _CH, _CH)])

        for b in range(nbuf):
            fire(b, b)

        def body(s, carry):
            for b in range(nbuf):
                j = nbuf * s + b
                wait(j, b)
                store(j, b)
                fire(jnp.minimum(j + nbuf, nch - 1), b)
            return carry

        lax.fori_loop(0, nch // nbuf, body, 0)
        for b in range(nbuf):
            wait(nch - 1, b)

    return k(x3, table)


def _re_body(i_ref, o_ref):
    for r in range(32):
        o_ref[r] = i_ref[0, pl.ds(r * 50, 50), :]


def _relayout(flat, nrow, ntok):
    grp = 32
    f3 = flat.reshape(nrow // grp, grp * ntok, EMB_DIM)
    return pl.pallas_call(
        _re_body,
        grid=(nrow // grp,),
        in_specs=[pl.BlockSpec((1, grp * ntok, EMB_DIM), lambda i: (i, 0, 0))],
        out_specs=pl.BlockSpec((grp, ntok, EMB_DIM), lambda i: (i, 0, 0)),
        out_shape=jax.ShapeDtypeStruct((nrow, ntok, EMB_DIM), jnp.float32),
    )(f3)


def kernel(x, emb0, emb1, emb2, emb3, W1, W2, W3):
    table = _build_table(emb0, emb1, emb2, emb3, W1, W2, W3)
    x3 = x.reshape(_NW, -1, _CH).astype(jnp.int32)
    out = _sc_lookup(x3, table)
    return _relayout(out, x.shape[0], x.shape[1])


# speedup vs baseline: 1.6639x; 1.1087x over previous
import functools

import jax
import jax.numpy as jnp
from jax import lax
from jax.experimental import pallas as pl
from jax.experimental.pallas import tpu as pltpu
from jax.experimental.pallas import tpu_sc as plsc

EMB_DIM = 128
_NC, _NS = 2, 16
_NW = _NC * _NS
_CH = 128

_BLK = 2048
_NB = (5, 5, 10, 30)


def _table_body(e0, e1, e2, e3, w1, w2, w3, o):
    i = pl.program_id(0)

    @pl.when(i < 5)
    def _():
        o[...] = e0[...]

    @pl.when((i >= 5) & (i < 10))
    def _():
        o[...] = jnp.dot(e1[...], w1[...], preferred_element_type=jnp.float32)

    @pl.when((i >= 10) & (i < 20))
    def _():
        o[...] = jnp.dot(e2[...], w2[...], preferred_element_type=jnp.float32)

    @pl.when(i >= 20)
    def _():
        o[...] = jnp.dot(e3[...], w3[...], preferred_element_type=jnp.float32)


def _build_table(emb0, emb1, emb2, emb3, w1, w2, w3):
    nrows = _BLK * sum(_NB)
    return pl.pallas_call(
        _table_body,
        grid=(sum(_NB),),
        in_specs=[
            pl.BlockSpec((_BLK, 128), lambda i: (jnp.clip(i, 0, 4), 0)),
            pl.BlockSpec((_BLK, 64), lambda i: (jnp.clip(i - 5, 0, 4), 0)),
            pl.BlockSpec((_BLK, 32), lambda i: (jnp.clip(i - 10, 0, 9), 0)),
            pl.BlockSpec((_BLK, 16), lambda i: (jnp.clip(i - 20, 0, 29), 0)),
            pl.BlockSpec((64, EMB_DIM), lambda i: (0, 0)),
            pl.BlockSpec((32, EMB_DIM), lambda i: (0, 0)),
            pl.BlockSpec((16, EMB_DIM), lambda i: (0, 0)),
        ],
        out_specs=pl.BlockSpec((_BLK, EMB_DIM), lambda i: (i, 0)),
        out_shape=jax.ShapeDtypeStruct((nrows, EMB_DIM), jnp.float32),
    )(emb0, emb1, emb2, emb3, w1, w2, w3)


def _sc_lookup(x3, table):
    nch = x3.shape[1]
    n_out = _NW * nch * _CH
    mesh = plsc.VectorSubcoreMesh(core_axis_name="c", subcore_axis_name="s")

    nbuf = 5
    @functools.partial(
        pl.kernel,
        out_type=jax.ShapeDtypeStruct((n_out, EMB_DIM), jnp.float32),
        mesh=mesh,
        scratch_types=[
            pltpu.VMEM((nch, _CH), jnp.int32),
            pltpu.VMEM((nbuf, _CH, EMB_DIM), jnp.float32),
            pltpu.SemaphoreType.DMA,
            pltpu.SemaphoreType.DMA,
            pltpu.SemaphoreType.DMA,
            pltpu.SemaphoreType.DMA,
            pltpu.SemaphoreType.DMA,
        ],
    )
    def k(x_hbm, table_hbm, out_hbm, idx_v, rows_v, *sems):
        wid = lax.axis_index("s") * _NC + lax.axis_index("c")
        cbase = wid * nch
        pltpu.sync_copy(x_hbm.at[wid], idx_v)

        def to_rows(r, carry):
            for c in range(_CH // 16):
                v = idx_v[r, pl.ds(c * 16, 16)]
                shift = (jnp.where(v > 10000, 240, 0)
                         + jnp.where(v > 20000, 240, 0)
                         + jnp.where(v > 40000, 480, 0))
                idx_v[r, pl.ds(c * 16, 16)] = jnp.where(v == 0, 10000,
                                                        v + shift)
            return carry

        lax.fori_loop(0, nch, to_rows, 0)

        def fire(j, b):
            pltpu.async_copy(table_hbm.at[idx_v.at[j]], rows_v.at[b], sems[b])

        def wait(j, b):
            pltpu.make_async_copy(table_hbm.at[idx_v.at[j]], rows_v.at[b],
                                  sems[b]).wait()

        def store(j, b):
            pltpu.sync_copy(rows_v.at[b],
                            out_hbm.at[pl.ds((cbase + j) * _CH, _CH)])

        for b in range(nbuf):
            fire(b, b)

        def body(s, carry):
            for b in range(nbuf):
                j = nbuf * s + b
                wait(j, b)
                store(j, b)
                fire(jnp.minimum(j + nbuf, nch - 1), b)
            return carry

        lax.fori_loop(0, nch // nbuf, body, 0)
        for b in range(nbuf):
            wait(nch - 1, b)

    return k(x3, table)


def _re_body(i_ref, o_ref):
    for r in range(64):
        o_ref[r] = i_ref[0, pl.ds(r * 50, 50), :]


def _relayout(flat, nrow, ntok):
    grp = 64
    f3 = flat.reshape(nrow // grp, grp * ntok, EMB_DIM)
    return pl.pallas_call(
        _re_body,
        grid=(nrow // grp,),
        in_specs=[pl.BlockSpec((1, grp * ntok, EMB_DIM), lambda i: (i, 0, 0))],
        out_specs=pl.BlockSpec((grp, ntok, EMB_DIM), lambda i: (i, 0, 0)),
        out_shape=jax.ShapeDtypeStruct((nrow, ntok, EMB_DIM), jnp.float32),
    )(f3)


def kernel(x, emb0, emb1, emb2, emb3, W1, W2, W3):
    table = _build_table(emb0, emb1, emb2, emb3, W1, W2, W3)
    x3 = x.reshape(_NW, -1, _CH).astype(jnp.int32)
    out = _sc_lookup(x3, table)
    return _relayout(out, x.shape[0], x.shape[1])


# speedup vs baseline: 1.7109x; 1.0283x over previous
import functools

import jax
import jax.numpy as jnp
from jax import lax
from jax.experimental import pallas as pl
from jax.experimental.pallas import tpu as pltpu
from jax.experimental.pallas import tpu_sc as plsc

EMB_DIM = 128
_NC, _NS = 2, 16
_NW = _NC * _NS
_CH = 128

_BLK = 2048
_NB = (5, 5, 10, 30)


def _table_body(e0, e1, e2, e3, w1, w2, w3, o):
    i = pl.program_id(0)

    @pl.when(i < 5)
    def _():
        o[...] = e0[...]

    @pl.when((i >= 5) & (i < 10))
    def _():
        o[...] = jnp.dot(e1[...], w1[...], preferred_element_type=jnp.float32)

    @pl.when((i >= 10) & (i < 20))
    def _():
        o[...] = jnp.dot(e2[...], w2[...], preferred_element_type=jnp.float32)

    @pl.when(i >= 20)
    def _():
        o[...] = jnp.dot(e3[...], w3[...], preferred_element_type=jnp.float32)


def _build_table(emb0, emb1, emb2, emb3, w1, w2, w3):
    nrows = _BLK * sum(_NB)
    return pl.pallas_call(
        _table_body,
        grid=(sum(_NB),),
        in_specs=[
            pl.BlockSpec((_BLK, 128), lambda i: (jnp.clip(i, 0, 4), 0)),
            pl.BlockSpec((_BLK, 64), lambda i: (jnp.clip(i - 5, 0, 4), 0)),
            pl.BlockSpec((_BLK, 32), lambda i: (jnp.clip(i - 10, 0, 9), 0)),
            pl.BlockSpec((_BLK, 16), lambda i: (jnp.clip(i - 20, 0, 29), 0)),
            pl.BlockSpec((64, EMB_DIM), lambda i: (0, 0)),
            pl.BlockSpec((32, EMB_DIM), lambda i: (0, 0)),
            pl.BlockSpec((16, EMB_DIM), lambda i: (0, 0)),
        ],
        out_specs=pl.BlockSpec((_BLK, EMB_DIM), lambda i: (i, 0)),
        out_shape=jax.ShapeDtypeStruct((nrows, EMB_DIM), jnp.float32),
    )(emb0, emb1, emb2, emb3, w1, w2, w3)


def _sc_lookup(x3, table):
    nch = x3.shape[1]
    n_out = _NW * nch * _CH
    mesh = plsc.VectorSubcoreMesh(core_axis_name="c", subcore_axis_name="s")

    nbuf = 5
    @functools.partial(
        pl.kernel,
        out_type=jax.ShapeDtypeStruct((n_out, EMB_DIM), jnp.float32),
        mesh=mesh,
        scratch_types=[
            pltpu.VMEM((nch, _CH), jnp.int32),
            pltpu.VMEM((nbuf, _CH, EMB_DIM), jnp.float32),
            pltpu.SemaphoreType.DMA,
            pltpu.SemaphoreType.DMA,
            pltpu.SemaphoreType.DMA,
            pltpu.SemaphoreType.DMA,
            pltpu.SemaphoreType.DMA,
        ],
    )
    def k(x_hbm, table_hbm, out_hbm, idx_v, rows_v, *sems):
        wid = lax.axis_index("s") * _NC + lax.axis_index("c")
        cbase = wid * nch
        pltpu.sync_copy(x_hbm.at[wid], idx_v)

        def to_rows(r, carry):
            for c in range(_CH // 16):
                v = idx_v[r, pl.ds(c * 16, 16)]
                shift = (jnp.where(v > 10000, 240, 0)
                         + jnp.where(v > 20000, 240, 0)
                         + jnp.where(v > 40000, 480, 0))
                idx_v[r, pl.ds(c * 16, 16)] = jnp.where(v == 0, 10000,
                                                        v + shift)
            return carry

        lax.fori_loop(0, nch, to_rows, 0)

        def fire(j, b):
            pltpu.async_copy(table_hbm.at[idx_v.at[j]], rows_v.at[b], sems[b])

        def wait(j, b):
            pltpu.make_async_copy(table_hbm.at[idx_v.at[j]], rows_v.at[b],
                                  sems[b]).wait()

        def store(j, b):
            pltpu.sync_copy(rows_v.at[b],
                            out_hbm.at[pl.ds((cbase + j) * _CH, _CH)])

        for b in range(nbuf):
            fire(b, b)

        def body(s, carry):
            for b in range(nbuf):
                j = nbuf * s + b
                wait(j, b)
                store(j, b)
                fire(jnp.minimum(j + nbuf, nch - 1), b)
            return carry

        lax.fori_loop(0, nch // nbuf, body, 0)
        for b in range(nbuf):
            wait(nch - 1, b)

    return k(x3, table)


def _re_body(i_ref, o_ref):
    for r in range(128):
        o_ref[r] = i_ref[0, pl.ds(r * 50, 50), :]


def _relayout(flat, nrow, ntok):
    grp = 128
    f3 = flat.reshape(nrow // grp, grp * ntok, EMB_DIM)
    return pl.pallas_call(
        _re_body,
        grid=(nrow // grp,),
        in_specs=[pl.BlockSpec((1, grp * ntok, EMB_DIM), lambda i: (i, 0, 0))],
        out_specs=pl.BlockSpec((grp, ntok, EMB_DIM), lambda i: (i, 0, 0)),
        out_shape=jax.ShapeDtypeStruct((nrow, ntok, EMB_DIM), jnp.float32),
    )(f3)


def kernel(x, emb0, emb1, emb2, emb3, W1, W2, W3):
    table = _build_table(emb0, emb1, emb2, emb3, W1, W2, W3)
    x3 = x.reshape(_NW, -1, _CH).astype(jnp.int32)
    out = _sc_lookup(x3, table)
    return _relayout(out, x.shape[0], x.shape[1])


# speedup vs baseline: 2.2032x; 1.2877x over previous
import functools

import jax
import jax.numpy as jnp
from jax import lax
from jax.experimental import pallas as pl
from jax.experimental.pallas import tpu as pltpu
from jax.experimental.pallas import tpu_sc as plsc

EMB_DIM = 128
_NC, _NS = 2, 16
_NW = _NC * _NS
_CH = 128

_BLK = 2048
_NB = (5, 5, 10, 30)


def _table_body(e0, e1, e2, e3, w1, w2, w3, o):
    i = pl.program_id(0)

    @pl.when(i < 5)
    def _():
        o[...] = e0[...]

    @pl.when((i >= 5) & (i < 10))
    def _():
        o[...] = jnp.dot(e1[...], w1[...], preferred_element_type=jnp.float32)

    @pl.when((i >= 10) & (i < 20))
    def _():
        o[...] = jnp.dot(e2[...], w2[...], preferred_element_type=jnp.float32)

    @pl.when(i >= 20)
    def _():
        o[...] = jnp.dot(e3[...], w3[...], preferred_element_type=jnp.float32)


def _build_table(emb0, emb1, emb2, emb3, w1, w2, w3):
    nrows = _BLK * sum(_NB)
    return pl.pallas_call(
        _table_body,
        grid=(sum(_NB),),
        in_specs=[
            pl.BlockSpec((_BLK, 128), lambda i: (jnp.clip(i, 0, 4), 0)),
            pl.BlockSpec((_BLK, 64), lambda i: (jnp.clip(i - 5, 0, 4), 0)),
            pl.BlockSpec((_BLK, 32), lambda i: (jnp.clip(i - 10, 0, 9), 0)),
            pl.BlockSpec((_BLK, 16), lambda i: (jnp.clip(i - 20, 0, 29), 0)),
            pl.BlockSpec((64, EMB_DIM), lambda i: (0, 0)),
            pl.BlockSpec((32, EMB_DIM), lambda i: (0, 0)),
            pl.BlockSpec((16, EMB_DIM), lambda i: (0, 0)),
        ],
        out_specs=pl.BlockSpec((_BLK, EMB_DIM), lambda i: (i, 0)),
        out_shape=jax.ShapeDtypeStruct((nrows, EMB_DIM), jnp.float32),
    )(emb0, emb1, emb2, emb3, w1, w2, w3)


def _sc_lookup(x3, table):
    nch = x3.shape[1]
    n_out = _NW * nch * _CH
    mesh = plsc.VectorSubcoreMesh(core_axis_name="c", subcore_axis_name="s")

    nbuf = 5
    @functools.partial(
        pl.kernel,
        out_type=jax.ShapeDtypeStruct((n_out, EMB_DIM), jnp.float32),
        mesh=mesh,
        scratch_types=[
            pltpu.VMEM((nch, _CH), jnp.int32),
            pltpu.VMEM((nbuf, _CH, EMB_DIM), jnp.float32),
            pltpu.SemaphoreType.DMA,
            pltpu.SemaphoreType.DMA,
            pltpu.SemaphoreType.DMA,
            pltpu.SemaphoreType.DMA,
            pltpu.SemaphoreType.DMA,
        ],
    )
    def k(x_hbm, table_hbm, out_hbm, idx_v, rows_v, *sems):
        wid = lax.axis_index("s") * _NC + lax.axis_index("c")
        cbase = wid * nch
        pltpu.sync_copy(x_hbm.at[wid], idx_v)

        def to_rows(r, carry):
            for c in range(_CH // 16):
                v = idx_v[r, pl.ds(c * 16, 16)]
                shift = (jnp.where(v > 10000, 240, 0)
                         + jnp.where(v > 20000, 240, 0)
                         + jnp.where(v > 40000, 480, 0))
                idx_v[r, pl.ds(c * 16, 16)] = jnp.where(v == 0, 10000,
                                                        v + shift)
            return carry

        lax.fori_loop(0, nch, to_rows, 0)

        def fire(j, b):
            pltpu.async_copy(table_hbm.at[idx_v.at[j]], rows_v.at[b], sems[b])

        def wait(j, b):
            pltpu.make_async_copy(table_hbm.at[idx_v.at[j]], rows_v.at[b],
                                  sems[b]).wait()

        def store(j, b):
            pltpu.sync_copy(rows_v.at[b],
                            out_hbm.at[pl.ds((cbase + j) * _CH, _CH)])

        for b in range(nbuf):
            fire(b, b)

        def body(s, carry):
            for b in range(nbuf):
                j = nbuf * s + b
                wait(j, b)
                store(j, b)
                fire(jnp.minimum(j + nbuf, nch - 1), b)
            return carry

        lax.fori_loop(0, nch // nbuf, body, 0)
        for b in range(nbuf):
            wait(nch - 1, b)

    return k(x3, table)


def _re_body(i_ref, o_ref):
    for r in range(128):
        o_ref[r] = i_ref[0, pl.ds(r * 50, 50), :]


def _relayout(flat, nrow, ntok):
    grp = 128
    f3 = flat.reshape(nrow // grp, grp * ntok, EMB_DIM)
    return pl.pallas_call(
        _re_body,
        grid=(nrow // grp,),
        in_specs=[pl.BlockSpec((1, grp * ntok, EMB_DIM), lambda i: (i, 0, 0))],
        out_specs=pl.BlockSpec((grp, ntok, EMB_DIM), lambda i: (i, 0, 0)),
        out_shape=jax.ShapeDtypeStruct((nrow, ntok, EMB_DIM), jnp.float32),
    )(f3)


def _sc_lookup_direct(x, table):
    nrow, ntok = x.shape
    rpw = nrow // _NW
    grp = 4
    ngr = rpw // grp
    mesh = plsc.VectorSubcoreMesh(core_axis_name="c", subcore_axis_name="s")

    @functools.partial(
        pl.kernel,
        out_type=jax.ShapeDtypeStruct((nrow, ntok, EMB_DIM), jnp.float32),
        mesh=mesh,
        scratch_types=[
            pltpu.VMEM((rpw, ntok), jnp.int32),
            pltpu.VMEM((rpw, ntok), jnp.int32),
            pltpu.VMEM((2 * grp, ntok, EMB_DIM), jnp.float32),
            pltpu.SemaphoreType.DMA,
            pltpu.SemaphoreType.DMA,
        ],
    )
    def k(x_hbm, table_hbm, out_hbm, xi_v, gi_v, stg_v, sem0, sem1):
        sems = (sem0, sem1)
        wid = lax.axis_index("s") * _NC + lax.axis_index("c")
        rbase = wid * rpw
        pltpu.sync_copy(x_hbm.at[pl.ds(rbase, rpw)], xi_v)

        def to_rows(r, carry):
            for c in (0, 16, 32, 34):
                v = xi_v[r, pl.ds(c, 16)]
                shift = (jnp.where(v > 10000, 240, 0)
                         + jnp.where(v > 20000, 240, 0)
                         + jnp.where(v > 40000, 480, 0))
                gi_v[r, pl.ds(c, 16)] = jnp.where(v == 0, 10000, v + shift)
            return carry

        lax.fori_loop(0, rpw, to_rows, 0)

        def fire_group(g, b):
            for i in range(grp):
                pltpu.async_copy(table_hbm.at[gi_v.at[grp * g + i]],
                                 stg_v.at[b * grp + i], sems[b])

        def wait_group(g, b):
            for i in range(grp):
                pltpu.make_async_copy(table_hbm.at[gi_v.at[grp * g + i]],
                                      stg_v.at[b * grp + i], sems[b]).wait()

        def store_group(g, b):
            pltpu.sync_copy(stg_v.at[pl.ds(b * grp, grp)],
                            out_hbm.at[pl.ds(rbase + grp * g, grp)])

        fire_group(0, 0)

        def body(s, carry):
            g0 = 2 * s
            g1 = g0 + 1
            fire_group(g1, 1)
            wait_group(g0, 0)
            store_group(g0, 0)
            fire_group(jnp.minimum(g0 + 2, ngr - 1), 0)
            wait_group(g1, 1)
            store_group(g1, 1)
            return carry

        lax.fori_loop(0, ngr // 2, body, 0)
        wait_group(ngr - 1, 0)

    return k(x, table)


def kernel(x, emb0, emb1, emb2, emb3, W1, W2, W3):
    table = _build_table(emb0, emb1, emb2, emb3, W1, W2, W3)
    return _sc_lookup_direct(x.astype(jnp.int32), table)
